# Initial kernel scaffold; baseline (speedup 1.0000x reference)
import jax
import jax.numpy as jnp
from jax.experimental import pallas as pl


def kernel(x, edge_index, ppi_list, idx, trainW, eps0, W1_0, b1_0, W2_0, b2_0, gamma0, beta0, eps1, W1_1, b1_1, W2_1, b2_1, gamma1, beta1, W_lin, b_lin, W_fc, b_fc):
    raise NotImplementedError("write your pallas kernel here")


# speedup vs baseline: 6.4292x; 6.4292x over previous
import functools

import jax
import jax.numpy as jnp
from jax import lax
from jax.experimental import pallas as pl
from jax.experimental.pallas import tpu as pltpu
from jax.experimental.pallas import tpu_sc as plsc

NC = 2
NS = 16
NW = NC * NS


@functools.partial(jax.jit, static_argnames=("n", "d", "e", "ch"))
def _sc_aggregate(x, src, dst, zeros, *, n, d, e, ch):
    per_w = e // NW
    n_ch = per_w // ch
    rows_a = (n // NS) // 8 * 8
    rows_last = n - (NS - 1) * rows_a

    src_r = src.reshape(NW, n_ch, ch)
    dst_r = dst.reshape(NW, n_ch, ch)

    mesh = plsc.VectorSubcoreMesh(core_axis_name="c", subcore_axis_name="s")

    @functools.partial(
        pl.kernel,
        mesh=mesh,
        out_type=[
            jax.ShapeDtypeStruct((n, d), jnp.float32),
            jax.ShapeDtypeStruct((n, d), jnp.float32),
        ],
        scratch_types=[
            pltpu.VMEM((n_ch, ch), jnp.int32),
            pltpu.VMEM((n_ch, ch), jnp.int32),
            pltpu.VMEM((ch, d), jnp.float32),
            pltpu.VMEM_SHARED((n, d), jnp.float32),
            pltpu.SemaphoreType.DMA,
        ],
    )
    def agg_k(x_hbm, src_hbm, dst_hbm, z_hbm, out0, out1, src_v, dst_v,
              rows_v, acc_sh, sem):
        c = lax.axis_index("c")
        s = lax.axis_index("s")
        wid = c * NS + s

        @pl.when(s < NS - 1)
        def _():
            pltpu.sync_copy(z_hbm.at[pl.ds(0, rows_a)],
                            acc_sh.at[pl.ds(s * rows_a, rows_a)])

        @pl.when(s == NS - 1)
        def _():
            pltpu.sync_copy(z_hbm,
                            acc_sh.at[pl.ds((NS - 1) * rows_a, rows_last)])

        pltpu.sync_copy(src_hbm.at[wid], src_v)
        pltpu.sync_copy(dst_hbm.at[wid], dst_v)
        plsc.subcore_barrier()

        def body(t, carry):
            pltpu.async_copy(x_hbm.at[src_v.at[t]], rows_v, sem).wait()
            pltpu.sync_copy(rows_v, acc_sh.at[dst_v.at[t]], add=True)
            return carry

        lax.fori_loop(0, n_ch, body, 0)
        plsc.subcore_barrier()

        sl_a = pl.ds(s * rows_a, rows_a)
        sl_l = pl.ds((NS - 1) * rows_a, rows_last)

        @pl.when((c == 0) & (s < NS - 1))
        def _():
            pltpu.sync_copy(acc_sh.at[sl_a], out0.at[sl_a])

        @pl.when((c == 0) & (s == NS - 1))
        def _():
            pltpu.sync_copy(acc_sh.at[sl_l], out0.at[sl_l])

        @pl.when((c == 1) & (s < NS - 1))
        def _():
            pltpu.sync_copy(acc_sh.at[sl_a], out1.at[sl_a])

        @pl.when((c == 1) & (s == NS - 1))
        def _():
            pltpu.sync_copy(acc_sh.at[sl_l], out1.at[sl_l])

    return agg_k(x, src_r, dst_r, zeros)


def _tc_mlp(x, p0, p1, eps, w1, b1, w2, b2, scale, beta, w3=None, b3=None,
            blk=1000):
    n, d = x.shape
    h = w1.shape[1]
    has_lin = w3 is not None
    grid = (n // blk,)

    def body(*refs):
        if has_lin:
            (eps_ref, x_ref, p0_ref, p1_ref, w1_ref, b1_ref, w2_ref, b2_ref,
             sc_ref, be_ref, w3_ref, b3_ref, o_ref) = refs
        else:
            (eps_ref, x_ref, p0_ref, p1_ref, w1_ref, b1_ref, w2_ref, b2_ref,
             sc_ref, be_ref, o_ref) = refs
        hh = (1.0 + eps_ref[0, 0]) * x_ref[...] + p0_ref[...] + p1_ref[...]
        hh = jnp.dot(hh, w1_ref[...], preferred_element_type=jnp.float32)
        hh = jnp.maximum(hh + b1_ref[...], 0.0)
        hh = jnp.dot(hh, w2_ref[...], preferred_element_type=jnp.float32)
        hh = jnp.maximum(hh + b2_ref[...], 0.0)
        hh = hh * sc_ref[...] + be_ref[...]
        if has_lin:
            hh = jnp.dot(hh, w3_ref[...], preferred_element_type=jnp.float32)
            hh = jnp.maximum(hh + b3_ref[...], 0.0)
        o_ref[...] = hh

    row_spec = pl.BlockSpec((blk, d), lambda i: (i, 0))
    w_spec = pl.BlockSpec((d, h), lambda i: (0, 0))
    v_spec = pl.BlockSpec((1, h), lambda i: (0, 0))
    s_spec = pl.BlockSpec((1, 1), lambda i: (0, 0))

    in_specs = [s_spec, row_spec, row_spec, row_spec, w_spec, v_spec, w_spec,
                v_spec, v_spec, v_spec]
    args = [eps.reshape(1, 1), x, p0, p1, w1, b1.reshape(1, h),
            w2, b2.reshape(1, h), scale.reshape(1, h), beta.reshape(1, h)]
    if has_lin:
        in_specs += [w_spec, v_spec]
        args += [w3, b3.reshape(1, h)]

    return pl.pallas_call(
        body,
        grid=grid,
        in_specs=in_specs,
        out_specs=pl.BlockSpec((blk, h), lambda i: (i, 0)),
        out_shape=jax.ShapeDtypeStruct((n, h), jnp.float32),
    )(*args)


@functools.partial(jax.jit, static_argnames=("n", "d", "b"))
def _sc_pair_gather(h, ppi0, ppi1, idx, *, n, d, b):
    cb = b // NW

    mesh = plsc.VectorSubcoreMesh(core_axis_name="c", subcore_axis_name="s")

    @functools.partial(
        pl.kernel,
        mesh=mesh,
        out_type=[
            jax.ShapeDtypeStruct((b, d), jnp.float32),
            jax.ShapeDtypeStruct((b, d), jnp.float32),
        ],
        scratch_types=[
            pltpu.VMEM((cb,), jnp.int32),
            pltpu.VMEM((cb,), jnp.int32),
            pltpu.VMEM((cb,), jnp.int32),
            pltpu.VMEM((cb, d), jnp.float32),
            pltpu.VMEM((cb, d), jnp.float32),
            pltpu.SemaphoreType.DMA,
        ],
    )
    def gather_k(h_hbm, p0_hbm, p1_hbm, idx_hbm, x1_hbm, x2_hbm, idx_v,
                 s0_v, s1_v, r0_v, r1_v, sem):
        c = lax.axis_index("c")
        s = lax.axis_index("s")
        wid = c * NS + s
        base = wid * cb

        pltpu.sync_copy(idx_hbm.at[pl.ds(base, cb)], idx_v)
        pltpu.async_copy(p0_hbm.at[idx_v], s0_v, sem).wait()
        pltpu.async_copy(p1_hbm.at[idx_v], s1_v, sem).wait()
        pltpu.async_copy(h_hbm.at[s0_v], r0_v, sem).wait()
        pltpu.async_copy(h_hbm.at[s1_v], r1_v, sem).wait()
        pltpu.sync_copy(r0_v, x1_hbm.at[pl.ds(base, cb)])
        pltpu.sync_copy(r1_v, x2_hbm.at[pl.ds(base, cb)])

    return gather_k(h, ppi0, ppi1, idx)


def _tc_fc(x1, x2, wa, wb, bf):
    b, d = x1.shape
    out = wa.shape[1]

    def body(x1_ref, x2_ref, wa_ref, wb_ref, bf_ref, o_ref):
        m = x1_ref[...] * x2_ref[...]
        sm = x1_ref[...] + x2_ref[...]
        acc = jnp.dot(m, wa_ref[...], preferred_element_type=jnp.float32)
        acc += jnp.dot(sm, wb_ref[...], preferred_element_type=jnp.float32)
        o_ref[...] = acc + bf_ref[...]

    return pl.pallas_call(
        body,
        out_shape=jax.ShapeDtypeStruct((b, out), jnp.float32),
    )(x1, x2, wa, wb, bf.reshape(1, out))


def kernel(x, edge_index, ppi_list, idx, trainW, eps0, W1_0, b1_0, W2_0, b2_0,
           gamma0, beta0, eps1, W1_1, b1_1, W2_1, b2_1, gamma1, beta1, W_lin,
           b_lin, W_fc, b_fc):
    n, d = x.shape
    e = edge_index.shape[1]
    b = idx.shape[0]
    h = W1_0.shape[1]

    src = edge_index[0].astype(jnp.int32)
    dst = edge_index[1].astype(jnp.int32)
    ppi0 = ppi_list[:, 0].astype(jnp.int32)
    ppi1 = ppi_list[:, 1].astype(jnp.int32)
    idx32 = idx.astype(jnp.int32)

    bn_inv = 1.0 / jnp.sqrt(1.0 + 1e-5)
    scale0 = gamma0 * bn_inv
    scale1 = gamma1 * bn_inv
    rows_a = (n // NS) // 8 * 8
    zeros = jnp.zeros((n - (NS - 1) * rows_a, d), jnp.float32)

    p0, p1 = _sc_aggregate(x, src, dst, zeros, n=n, d=d, e=e, ch=80)
    h1 = _tc_mlp(x, p0, p1, eps0, W1_0, b1_0, W2_0, b2_0, scale0, beta0)
    q0, q1 = _sc_aggregate(h1, src, dst, zeros, n=n, d=d, e=e, ch=80)
    h3 = _tc_mlp(h1, q0, q1, eps1, W1_1, b1_1, W2_1, b2_1, scale1, beta1,
                 w3=W_lin, b3=b_lin)

    x1, x2 = _sc_pair_gather(h3, ppi0, ppi1, idx32, n=n, d=d, b=b)
    out = _tc_fc(x1, x2, W_fc[:h], W_fc[h:], b_fc)
    return (out, jnp.float32(0.0))
